# baseline (speedup 1.0000x reference)
import jax
import jax.numpy as jnp
from jax import lax
from jax.experimental import pallas as pl
from jax.experimental.pallas import tpu as pltpu
from jax.experimental.pallas import tpu_sc as plsc

NC = 2
NS = 16
NW = NC * NS

B_CTX = 4096
B_LIST = 50
DIM = 64
LPAD = 56
DPAD = 128
CTX_PER_W = B_CTX // NW
EX_PER_W = CTX_PER_W * B_LIST
NBUF = 2
BB = 8
CHUNK = BB * B_LIST
EX_CHUNKS = CTX_PER_W // BB


def _sc_body(table, ctx_ids, ex_ids, ctx_out, ex_out,
             cidx, crows, eidx, bufs, gsems, ssems, csem):
    wid = lax.axis_index("s") * NC + lax.axis_index("c")
    batch0 = wid * CTX_PER_W

    pltpu.sync_copy(ctx_ids.at[pl.ds(batch0, CTX_PER_W)], cidx)
    ctx_gather = pltpu.async_copy(table.at[cidx], crows, csem)

    pltpu.sync_copy(ex_ids.at[wid], eidx)

    def gather_start(c, b):
        pltpu.async_copy(
            table.at[eidx.at[pl.ds(c * CHUNK, CHUNK)]], bufs.at[b],
            gsems.at[b])

    def gather_wait(c, b):
        pltpu.make_async_copy(
            table.at[eidx.at[pl.ds(c * CHUNK, CHUNK)]], bufs.at[b],
            gsems.at[b]).wait()

    def store_parts(c, b):
        for i in range(BB):
            src = bufs.at[b].at[pl.ds(i * B_LIST, B_LIST)]
            dst = ex_out.at[batch0 + c * BB + i, pl.ds(0, B_LIST),
                            pl.ds(0, DIM)]
            yield src, dst

    def store_start(c, b):
        for src, dst in store_parts(c, b):
            pltpu.async_copy(src, dst, ssems.at[b])

    def store_wait(c, b):
        for src, dst in store_parts(c, b):
            pltpu.make_async_copy(src, dst, ssems.at[b]).wait()

    for b in range(NBUF):
        gather_start(b, b)

    ctx_gather.wait()
    pltpu.sync_copy(
        crows, ctx_out.at[pl.ds(batch0, CTX_PER_W), pl.ds(0, DIM)])

    @pl.loop(0, EX_CHUNKS - NBUF, step=NBUF)
    def _(j):
        for b in range(NBUF):
            gather_wait(j + b, b)
            store_start(j + b, b)
        for b in range(NBUF):
            store_wait(j + b, b)
            gather_start(j + b + NBUF, b)

    for b in range(NBUF):
        c = EX_CHUNKS - NBUF + b
        gather_wait(c, b)
        store_start(c, b)
    for b in range(NBUF):
        store_wait(EX_CHUNKS - NBUF + b, b)


@jax.jit
def _encode(table, ctx_ids, ex_ids):
    mesh = plsc.VectorSubcoreMesh(core_axis_name="c", subcore_axis_name="s")
    ctx_out, ex_out = pl.kernel(
        _sc_body,
        out_type=(
            jax.ShapeDtypeStruct((B_CTX, DPAD), jnp.float32),
            jax.ShapeDtypeStruct((B_CTX, LPAD, DPAD), jnp.float32),
        ),
        mesh=mesh,
        compiler_params=pltpu.CompilerParams(use_tc_tiling_on_sc=False),
        scratch_types=[
            pltpu.VMEM((CTX_PER_W,), jnp.int32),
            pltpu.VMEM((CTX_PER_W, DIM), jnp.float32),
            pltpu.VMEM((EX_PER_W,), jnp.int32),
            pltpu.VMEM((NBUF, CHUNK, DIM), jnp.float32),
            pltpu.SemaphoreType.DMA((NBUF,)),
            pltpu.SemaphoreType.DMA((NBUF,)),
            pltpu.SemaphoreType.DMA,
        ],
    )(table, ctx_ids, ex_ids)
    return ctx_out, ex_out


def kernel(context_table, example_table, context_ids, example_ids):
    B, L = example_ids.shape
    n_ctx = context_table.shape[0]
    table = jnp.concatenate([context_table, example_table], axis=0)
    ctx_ids = jnp.asarray(context_ids, jnp.int32)
    ex_ids = (jnp.asarray(example_ids, jnp.int32) + n_ctx).reshape(
        NW, EX_PER_W)
    ctx_out, ex_out = _encode(table, ctx_ids, ex_ids)
    return ctx_out[:, :DIM], ex_out[:, :L, :DIM]


# speedup vs baseline: 1.3313x; 1.3313x over previous
import jax
import jax.numpy as jnp
from jax import lax
from jax.experimental import pallas as pl
from jax.experimental.pallas import tpu as pltpu
from jax.experimental.pallas import tpu_sc as plsc

NC = 2
NS = 16
NW = NC * NS

B_CTX = 4096
B_LIST = 50
DIM = 64
LPAD = 56
DPAD = 128
CTX_PER_W = B_CTX // NW
EX_PER_W = CTX_PER_W * B_LIST
NBUF = 4
BB = 4
CHUNK = BB * B_LIST
EX_CHUNKS = CTX_PER_W // BB


def _sc_body(ctx_table, ex_table, ctx_ids, ex_ids, ctx_out, ex_out,
             cidx, crows, eidx, bufs, gsems, ssems, csem):
    wid = lax.axis_index("s") * NC + lax.axis_index("c")
    batch0 = wid * CTX_PER_W

    pltpu.sync_copy(ctx_ids.at[pl.ds(batch0, CTX_PER_W)], cidx)
    ctx_gather = pltpu.async_copy(ctx_table.at[cidx], crows, csem)

    pltpu.sync_copy(ex_ids.at[wid], eidx)

    def gather_start(c, b):
        pltpu.async_copy(
            ex_table.at[eidx.at[pl.ds(c * CHUNK, CHUNK)]], bufs.at[b],
            gsems.at[b])

    def gather_wait(c, b):
        pltpu.make_async_copy(
            ex_table.at[eidx.at[pl.ds(c * CHUNK, CHUNK)]], bufs.at[b],
            gsems.at[b]).wait()

    def store_parts(c, b):
        for i in range(BB):
            src = bufs.at[b].at[pl.ds(i * B_LIST, B_LIST)]
            dst = ex_out.at[batch0 + c * BB + i, pl.ds(0, B_LIST),
                            pl.ds(0, DIM)]
            yield src, dst

    def store_start(c, b):
        for src, dst in store_parts(c, b):
            pltpu.async_copy(src, dst, ssems.at[b])

    def store_wait(c, b):
        for src, dst in store_parts(c, b):
            pltpu.make_async_copy(src, dst, ssems.at[b]).wait()

    for b in range(NBUF):
        gather_start(b, b)

    ctx_gather.wait()
    pltpu.sync_copy(
        crows, ctx_out.at[pl.ds(batch0, CTX_PER_W), pl.ds(0, DIM)])

    @pl.loop(0, EX_CHUNKS - NBUF, step=NBUF)
    def _(j):
        for b in range(NBUF):
            gather_wait(j + b, b)
            store_start(j + b, b)
        for b in range(NBUF):
            store_wait(j + b, b)
            gather_start(j + b + NBUF, b)

    for b in range(NBUF):
        c = EX_CHUNKS - NBUF + b
        gather_wait(c, b)
        store_start(c, b)
    for b in range(NBUF):
        store_wait(EX_CHUNKS - NBUF + b, b)


@jax.jit
def _encode(ctx_table, ex_table, ctx_ids, ex_ids):
    mesh = plsc.VectorSubcoreMesh(core_axis_name="c", subcore_axis_name="s")
    ctx_out, ex_out = pl.kernel(
        _sc_body,
        out_type=(
            jax.ShapeDtypeStruct((B_CTX, DPAD), jnp.float32),
            jax.ShapeDtypeStruct((B_CTX, LPAD, DPAD), jnp.float32),
        ),
        mesh=mesh,
        compiler_params=pltpu.CompilerParams(use_tc_tiling_on_sc=False),
        scratch_types=[
            pltpu.VMEM((CTX_PER_W,), jnp.int32),
            pltpu.VMEM((CTX_PER_W, DIM), jnp.float32),
            pltpu.VMEM((EX_PER_W,), jnp.int32),
            pltpu.VMEM((NBUF, CHUNK, DIM), jnp.float32),
            pltpu.SemaphoreType.DMA((NBUF,)),
            pltpu.SemaphoreType.DMA((NBUF,)),
            pltpu.SemaphoreType.DMA,
        ],
    )(ctx_table, ex_table, ctx_ids, ex_ids)
    return ctx_out, ex_out


def kernel(context_table, example_table, context_ids, example_ids):
    B, L = example_ids.shape
    ctx_ids = jnp.asarray(context_ids, jnp.int32)
    ex_ids = jnp.asarray(example_ids, jnp.int32).reshape(NW, EX_PER_W)
    ctx_out, ex_out = _encode(context_table, example_table, ctx_ids, ex_ids)
    return ctx_out[:, :DIM], ex_out[:, :L, :DIM]
